# Initial kernel scaffold; baseline (speedup 1.0000x reference)
import jax
import jax.numpy as jnp
from jax.experimental import pallas as pl


def kernel(xr, xi, skip0, skip1, w_re, w_im, Wrr, Wii, Wri, Br, Bi):
    raise NotImplementedError("write your pallas kernel here")


# speedup vs baseline: 5.4193x; 5.4193x over previous
import functools

import jax
import jax.numpy as jnp
from jax import lax
from jax.experimental import pallas as pl
from jax.experimental.pallas import tpu as pltpu


def _conv_stats_kernel(x0_ref, x1_ref, w_ref, c_ref, stats_ref, *,
                       tr, shifts, ntc, wp, hp, wo, ho):
    xcat = jnp.concatenate([x0_ref[...], x1_ref[...]], axis=0)
    lhs = jnp.concatenate([xcat[s:s + tr] for s in shifts], axis=1)
    acc = jnp.dot(lhs, w_ref[...], preferred_element_type=jnp.float32)
    c_ref[...] = acc.astype(jnp.bfloat16)

    t0 = (pl.program_id(0) * ntc + pl.program_id(1)) * tr
    bw = jnp.mod(t0, wp)
    q0 = t0 // wp
    bh = jnp.mod(q0, hp)
    r = lax.broadcasted_iota(jnp.int32, (tr, 128), 0).astype(jnp.float32)
    t = r + bw.astype(jnp.float32)
    f1 = jnp.floor((t + 0.5) * (1.0 / wp))
    ow = t - f1 * wp
    oh_un = f1 + bh.astype(jnp.float32)
    f2 = jnp.floor((oh_un + 0.5) * (1.0 / hp))
    oh = oh_un - f2 * hp
    mk = jnp.where((ow < wo) & (oh < ho), 1.0, 0.0)

    cm = acc * mk
    s1 = jnp.sum(cm, axis=0, keepdims=True)
    s2 = jnp.sum(acc * cm, axis=0, keepdims=True)
    s3 = jnp.sum(cm[:, :64] * acc[:, 64:], axis=0, keepdims=True)

    @pl.when(pl.program_id(1) == 0)
    def _():
        stats_ref[...] = jnp.zeros_like(stats_ref)

    stats_ref[0:1, :] += s1
    stats_ref[1:2, :] += s2
    stats_ref[2:3, :64] += s3


def _coef_kernel(stats_ref, par_ref, coef_ref, *, inv_m, eps):
    s1 = stats_ref[0:1, :] + stats_ref[8:9, :]
    s2 = stats_ref[1:2, :] + stats_ref[9:10, :]
    s3 = stats_ref[2:3, :64] + stats_ref[10:11, :64]

    mr = s1[:, :64] * inv_m
    mi = s1[:, 64:] * inv_m
    vrr = s2[:, :64] * inv_m - mr * mr + eps
    vii = s2[:, 64:] * inv_m - mi * mi + eps
    vri = s3 * inv_m - mr * mi

    tau = vrr + vii
    delta = vrr * vii - vri * vri
    s = jnp.sqrt(delta)
    rst = lax.rsqrt(delta * (tau + 2.0 * s))
    urr = (s + vii) * rst
    uii = (s + vrr) * rst
    uri = -vri * rst

    wrr = par_ref[0:1, :64]
    wri = par_ref[1:2, :64]
    wii = par_ref[2:3, :64]
    br = par_ref[3:4, :64]
    bi = par_ref[4:5, :64]

    zrr = wrr * urr + wri * uri
    zri = wrr * uri + wri * uii
    zir = wri * urr + wii * uri
    zii = wri * uri + wii * uii

    coef_ref[...] = jnp.zeros_like(coef_ref)
    coef_ref[0:1, :64] = zrr
    coef_ref[0:1, 64:] = zii
    coef_ref[1:2, :64] = zri
    coef_ref[1:2, 64:] = zir
    coef_ref[2:3, :64] = br - zrr * mr - zri * mi
    coef_ref[2:3, 64:] = bi - zir * mr - zii * mi


def _apply_kernel(c_ref, coef_ref, y_ref, *, slope):
    cf = c_ref[...].astype(jnp.float32)
    swapped = jnp.concatenate([cf[:, 64:], cf[:, :64]], axis=1)
    y = (coef_ref[0:1, :] * cf + coef_ref[1:2, :] * swapped
         + coef_ref[2:3, :])
    y_ref[...] = jnp.where(y >= 0, y, slope * y).astype(jnp.bfloat16)


def kernel(xr, xi, skip0, skip1, w_re, w_im, Wrr, Wii, Wri, Br, Bi):
    n, c1, H, W = xr.shape
    cin = c1 + skip0.shape[1]
    cout = w_re.shape[1]
    cf = 2 * cin
    kh = kw = 3
    slope, eps = 0.1, 1e-5

    ho = (H - 1) * 2 + kh
    wo = (W - 1) * 1 + kw
    hp = ho + kh - 1
    wp = wo + kw - 1

    xre = jnp.concatenate([xr, skip0], axis=1).transpose(0, 2, 3, 1)
    xim = jnp.concatenate([xi, skip1], axis=1).transpose(0, 2, 3, 1)
    xf = jnp.concatenate([xre, xim], axis=3).astype(jnp.bfloat16)

    z = jnp.zeros((n, hp, wp, cf), jnp.bfloat16)
    z = z.at[:, 2:2 * H + 1:2, 2:2 + W, :].set(xf)

    mflat = n * hp * wp
    tr = 512
    nt = -(-mflat // tr)
    nt += nt % 2
    mpad = nt * tr
    xflat = jnp.zeros((mpad + tr, cf), jnp.bfloat16)
    xflat = xflat.at[:mflat].set(z.reshape(mflat, cf))

    wre_f = jnp.flip(w_re, axis=(2, 3)).transpose(2, 3, 0, 1)
    wim_f = jnp.flip(w_im, axis=(2, 3)).transpose(2, 3, 0, 1)
    top = jnp.concatenate([wre_f, wim_f], axis=3)
    bot = jnp.concatenate([-wim_f, wre_f], axis=3)
    wfull = jnp.concatenate([top, bot], axis=2).reshape(
        kh * kw * cf, 2 * cout).astype(jnp.bfloat16)

    ntc = nt // 2
    shifts = tuple(i * wp + j for i in range(kh) for j in range(kw))

    conv_fn = functools.partial(_conv_stats_kernel, tr=tr, shifts=shifts,
                                ntc=ntc, wp=wp, hp=hp, wo=wo, ho=ho)
    cbuf, stats = pl.pallas_call(
        conv_fn,
        grid=(2, ntc),
        in_specs=[
            pl.BlockSpec((tr, cf), lambda c, i: (c * ntc + i, 0)),
            pl.BlockSpec((tr, cf), lambda c, i: (c * ntc + i + 1, 0)),
            pl.BlockSpec((kh * kw * cf, 2 * cout), lambda c, i: (0, 0)),
        ],
        out_specs=(
            pl.BlockSpec((tr, 2 * cout), lambda c, i: (c * ntc + i, 0)),
            pl.BlockSpec((8, 2 * cout), lambda c, i: (c, 0)),
        ),
        out_shape=(
            jax.ShapeDtypeStruct((mpad, 2 * cout), jnp.bfloat16),
            jax.ShapeDtypeStruct((16, 2 * cout), jnp.float32),
        ),
        compiler_params=pltpu.CompilerParams(
            dimension_semantics=("parallel", "arbitrary"),
            vmem_limit_bytes=48 * 1024 * 1024),
    )(xflat, xflat, wfull)

    params = jnp.zeros((8, 2 * cout), jnp.float32)
    params = params.at[0, :cout].set(Wrr[0])
    params = params.at[1, :cout].set(Wri[0])
    params = params.at[2, :cout].set(Wii[0])
    params = params.at[3, :cout].set(Br[0])
    params = params.at[4, :cout].set(Bi[0])

    m = n * ho * wo
    coef_fn = functools.partial(_coef_kernel, inv_m=1.0 / float(m), eps=eps)
    coef = pl.pallas_call(
        coef_fn,
        grid=(1,),
        in_specs=[pl.BlockSpec((16, 2 * cout), lambda i: (0, 0)),
                  pl.BlockSpec((8, 2 * cout), lambda i: (0, 0))],
        out_specs=pl.BlockSpec((8, 2 * cout), lambda i: (0, 0)),
        out_shape=jax.ShapeDtypeStruct((8, 2 * cout), jnp.float32),
        compiler_params=pltpu.CompilerParams(
            dimension_semantics=("arbitrary",)),
    )(stats, params)

    apply_fn = functools.partial(_apply_kernel, slope=slope)
    y = pl.pallas_call(
        apply_fn,
        grid=(2, ntc),
        in_specs=[
            pl.BlockSpec((tr, 2 * cout), lambda c, i: (c * ntc + i, 0)),
            pl.BlockSpec((8, 2 * cout), lambda c, i: (0, 0)),
        ],
        out_specs=pl.BlockSpec((tr, 2 * cout), lambda c, i: (c * ntc + i, 0)),
        out_shape=jax.ShapeDtypeStruct((mpad, 2 * cout), jnp.bfloat16),
        compiler_params=pltpu.CompilerParams(
            dimension_semantics=("parallel", "arbitrary"),
            vmem_limit_bytes=48 * 1024 * 1024),
    )(cbuf, coef)

    yv = y[:mflat].reshape(n, hp, wp, 2 * cout)[:, :ho, :wo, :]
    yr = yv[..., :cout].astype(jnp.float32).transpose(0, 3, 1, 2)
    yi = yv[..., cout:].astype(jnp.float32).transpose(0, 3, 1, 2)
    return yr, yi
